# baseline (speedup 1.0000x reference)
import functools
import math

import jax
import jax.numpy as jnp
from jax import lax
from jax.experimental import pallas as pl
from jax.experimental.pallas import tpu as pltpu
from jax.experimental.pallas import tpu_sc as plsc

_NC = 2
_NS = 16
_NW = _NC * _NS
_CHUNK = 128


def _sc_gather(weight, bias, nlp, target, samples, log_ns):
    V, D = weight.shape
    B = target.shape[0]
    S = samples.shape[0]
    mesh = plsc.VectorSubcoreMesh(core_axis_name="c", subcore_axis_name="s")

    @functools.partial(
        pl.kernel,
        mesh=mesh,
        out_type=[
            jax.ShapeDtypeStruct((B, D), jnp.float32),
            jax.ShapeDtypeStruct((S, D), jnp.float32),
            jax.ShapeDtypeStruct((B,), jnp.float32),
            jax.ShapeDtypeStruct((S,), jnp.float32),
        ],
        scratch_types=[
            pltpu.VMEM((_CHUNK,), jnp.int32),
            pltpu.VMEM((_CHUNK, D), jnp.float32),
            pltpu.VMEM((_CHUNK,), jnp.float32),
            pltpu.VMEM((_CHUNK,), jnp.float32),
            pltpu.VMEM((_CHUNK,), jnp.float32),
            pltpu.SemaphoreType.DMA,
        ],
        compiler_params=pltpu.CompilerParams(use_tc_tiling_on_sc=False),
    )
    def gather_kernel(weight_hbm, bias_hbm, nlp_hbm, target_hbm, samples_hbm,
                      trows_hbm, srows_hbm, toff_hbm, soff_hbm,
                      idx_v, rows_v, b_v, n_v, o_v, sem):
        wid = lax.axis_index("s") * _NC + lax.axis_index("c")

        def do_chunk(idx_hbm, rows_out, off_out, base):
            pltpu.sync_copy(idx_hbm.at[pl.ds(base, _CHUNK)], idx_v)
            pltpu.async_copy(weight_hbm.at[idx_v], rows_v, sem).wait()
            pltpu.async_copy(bias_hbm.at[idx_v], b_v, sem).wait()
            pltpu.async_copy(nlp_hbm.at[idx_v], n_v, sem).wait()
            for k in range(_CHUNK // 16):
                sl = pl.ds(k * 16, 16)
                o_v[sl] = b_v[sl] - (n_v[sl] + log_ns)
            pltpu.sync_copy(rows_v, rows_out.at[pl.ds(base, _CHUNK)])
            pltpu.sync_copy(o_v, off_out.at[pl.ds(base, _CHUNK)])

        for c in range(B // (_NW * _CHUNK)):
            do_chunk(target_hbm, trows_hbm, toff_hbm,
                     wid * (B // _NW) + c * _CHUNK)
        for c in range(S // (_NW * _CHUNK)):
            do_chunk(samples_hbm, srows_hbm, soff_hbm,
                     wid * (S // _NW) + c * _CHUNK)

    return gather_kernel(weight, bias, nlp, target, samples)


_LOG2E = 1.4426950408889634
_LN2 = 0.6931471805599453


def _softplus_neg_abs(x):
    p = jnp.exp2(jnp.abs(x) * (-_LOG2E))
    return _LN2 * jnp.log2(1.0 + p)


def _tc_loss_body(d_ref, te_ref, to_ref, se_ref, so_ref, out_ref):
    d = d_ref[...]
    tl = jnp.sum(d * te_ref[...], axis=1, keepdims=True) + to_ref[...]
    sl = lax.dot_general(d, se_ref[...],
                         dimension_numbers=(((1,), (1,)), ((), ())),
                         preferred_element_type=jnp.float32)
    sl = sl + so_ref[...]
    loss_t = jnp.maximum(tl, 0.0) - tl + _softplus_neg_abs(tl)
    loss_s = jnp.maximum(sl, 0.0) + _softplus_neg_abs(sl)
    out_ref[...] = jnp.concatenate([loss_t, loss_s], axis=1)


def _tc_loss(data, true_rows, samp_rows, true_off, samp_off, bt=256):
    B, D = data.shape
    S = samp_rows.shape[0]
    return pl.pallas_call(
        _tc_loss_body,
        grid=(B // bt,),
        in_specs=[
            pl.BlockSpec((bt, D), lambda i: (i, 0)),
            pl.BlockSpec((bt, D), lambda i: (i, 0)),
            pl.BlockSpec((bt, 1), lambda i: (i, 0)),
            pl.BlockSpec((S, D), lambda i: (0, 0)),
            pl.BlockSpec((1, S), lambda i: (0, 0)),
        ],
        out_specs=pl.BlockSpec((bt, S + 1), lambda i: (i, 0)),
        out_shape=jax.ShapeDtypeStruct((B, S + 1), jnp.float32),
        compiler_params=pltpu.CompilerParams(
            dimension_semantics=("arbitrary",),
        ),
    )(data, true_rows, true_off.reshape(B, 1), samp_rows,
      samp_off.reshape(1, S))


def kernel(data, target, samples, weight, bias, noise_log_probs):
    log_ns = math.log(samples.shape[0])
    true_rows, samp_rows, true_off, samp_off = _sc_gather(
        weight, bias, noise_log_probs,
        target.astype(jnp.int32), samples.astype(jnp.int32), log_ns)
    return _tc_loss(data, true_rows, samp_rows, true_off, samp_off)


# speedup vs baseline: 1.1836x; 1.1836x over previous
import functools
import math

import jax
import jax.numpy as jnp
from jax import lax
from jax.experimental import pallas as pl
from jax.experimental.pallas import tpu as pltpu
from jax.experimental.pallas import tpu_sc as plsc

_NC = 2
_NS = 16
_NW = _NC * _NS
_CHUNK = 128


def _sc_gather(weight, bias, nlp, target, samples, log_ns):
    V, D = weight.shape
    B = target.shape[0]
    S = samples.shape[0]
    mesh = plsc.VectorSubcoreMesh(core_axis_name="c", subcore_axis_name="s")

    @functools.partial(
        pl.kernel,
        mesh=mesh,
        out_type=[
            jax.ShapeDtypeStruct((B, D), jnp.float32),
            jax.ShapeDtypeStruct((S, D), jnp.float32),
            jax.ShapeDtypeStruct((B,), jnp.float32),
            jax.ShapeDtypeStruct((S,), jnp.float32),
        ],
        scratch_types=[
            pltpu.VMEM((_CHUNK,), jnp.int32),
            pltpu.VMEM((_CHUNK, D), jnp.float32),
            pltpu.VMEM((_CHUNK,), jnp.float32),
            pltpu.VMEM((_CHUNK,), jnp.float32),
            pltpu.VMEM((_CHUNK,), jnp.float32),
            pltpu.SemaphoreType.DMA,
        ],
        compiler_params=pltpu.CompilerParams(use_tc_tiling_on_sc=False),
    )
    def gather_kernel(weight_hbm, bias_hbm, nlp_hbm, target_hbm, samples_hbm,
                      trows_hbm, srows_hbm, toff_hbm, soff_hbm,
                      idx_v, rows_v, b_v, n_v, o_v, sem):
        wid = lax.axis_index("s") * _NC + lax.axis_index("c")

        def do_chunk(idx_hbm, rows_out, off_out, base):
            pltpu.sync_copy(idx_hbm.at[pl.ds(base, _CHUNK)], idx_v)
            pltpu.async_copy(weight_hbm.at[idx_v], rows_v, sem).wait()
            pltpu.async_copy(bias_hbm.at[idx_v], b_v, sem).wait()
            pltpu.async_copy(nlp_hbm.at[idx_v], n_v, sem).wait()
            for k in range(_CHUNK // 16):
                sl = pl.ds(k * 16, 16)
                o_v[sl] = b_v[sl] - (n_v[sl] + log_ns)
            pltpu.sync_copy(rows_v, rows_out.at[pl.ds(base, _CHUNK)])
            pltpu.sync_copy(o_v, off_out.at[pl.ds(base, _CHUNK)])

        for c in range(B // (_NW * _CHUNK)):
            do_chunk(target_hbm, trows_hbm, toff_hbm,
                     wid * (B // _NW) + c * _CHUNK)
        for c in range(S // (_NW * _CHUNK)):
            do_chunk(samples_hbm, srows_hbm, soff_hbm,
                     wid * (S // _NW) + c * _CHUNK)

    return gather_kernel(weight, bias, nlp, target, samples)


_LOG2E = 1.4426950408889634
_LN2 = 0.6931471805599453


def _softplus_neg_abs(x):
    p = jnp.exp2(jnp.abs(x) * (-_LOG2E))
    return _LN2 * jnp.log2(1.0 + p)


def _tc_loss_T_body(se_ref, dt_ref, tt_ref, to_ref, so_ref, out_ref):
    j = pl.program_id(0)
    mm = lax.dot_general(se_ref[...], dt_ref[...],
                         dimension_numbers=(((1,), (0,)), ((), ())),
                         preferred_element_type=jnp.float32)
    sl = mm + so_ref[...]
    out_ref[...] = jnp.maximum(sl, 0.0) + _softplus_neg_abs(sl)

    @pl.when(j == 0)
    def _():
        tl = jnp.sum(dt_ref[...] * tt_ref[...], axis=0, keepdims=True)
        tl = tl + to_ref[...]
        out_ref[0:1, :] = jnp.maximum(tl, 0.0) - tl + _softplus_neg_abs(tl)


def _tc_loss_T(dataT, srows_p, soff_p, trowsT, toff_row, bt=512):
    D, B = dataT.shape
    S1 = srows_p.shape[0]
    grid = (pl.cdiv(S1, bt),)
    return pl.pallas_call(
        _tc_loss_T_body,
        grid=grid,
        in_specs=[
            pl.BlockSpec((bt, D), lambda j: (j, 0)),
            pl.BlockSpec((D, B), lambda j: (0, 0)),
            pl.BlockSpec((D, B), lambda j: (0, 0)),
            pl.BlockSpec((1, B), lambda j: (0, 0)),
            pl.BlockSpec((bt, 1), lambda j: (j, 0)),
        ],
        out_specs=pl.BlockSpec((bt, B), lambda j: (j, 0)),
        out_shape=jax.ShapeDtypeStruct((S1, B), jnp.float32),
        compiler_params=pltpu.CompilerParams(
            dimension_semantics=("arbitrary",),
        ),
    )(srows_p, dataT, trowsT, toff_row, soff_p)


def kernel(data, target, samples, weight, bias, noise_log_probs):
    B = data.shape[0]
    S = samples.shape[0]
    log_ns = math.log(S)
    true_rows, samp_rows, true_off, samp_off = _sc_gather(
        weight, bias, noise_log_probs,
        target.astype(jnp.int32), samples.astype(jnp.int32), log_ns)
    srows_p = jnp.pad(samp_rows, ((1, 0), (0, 0)))
    soff_p = jnp.pad(samp_off, (1, 0)).reshape(S + 1, 1)
    out_T = _tc_loss_T(data.T, srows_p, soff_p, true_rows.T,
                       true_off.reshape(1, B))
    return out_T.T


# speedup vs baseline: 2.3688x; 2.0014x over previous
import functools
import math

import jax
import jax.numpy as jnp
from jax import lax
from jax.experimental import pallas as pl
from jax.experimental.pallas import tpu as pltpu
from jax.experimental.pallas import tpu_sc as plsc

_NC = 2
_NS = 16
_NW = _NC * _NS
_CHUNK = 128


def _expand_body(wt_ref, out_ref):
    D = wt_ref.shape[0]
    out_ref[:, :D] = wt_ref[...].T


def _tc_expand(weightT, nc=16384):
    D, V = weightT.shape
    return pl.pallas_call(
        _expand_body,
        grid=(pl.cdiv(V, nc),),
        in_specs=[pl.BlockSpec((D, nc), lambda j: (0, j))],
        out_specs=pl.BlockSpec((nc, 2 * D), lambda j: (j, 0)),
        out_shape=jax.ShapeDtypeStruct((V, 2 * D), jnp.float32),
        compiler_params=pltpu.CompilerParams(
            dimension_semantics=("arbitrary",),
        ),
    )(weightT)


def _sc_gather(w2, bias, nlp, target, samples, log_ns):
    V, D2 = w2.shape
    B = target.shape[0]
    S = samples.shape[0]
    mesh = plsc.VectorSubcoreMesh(core_axis_name="c", subcore_axis_name="s")

    @functools.partial(
        pl.kernel,
        mesh=mesh,
        out_type=[
            jax.ShapeDtypeStruct((B, D2), jnp.float32),
            jax.ShapeDtypeStruct((S, D2), jnp.float32),
            jax.ShapeDtypeStruct((B,), jnp.float32),
            jax.ShapeDtypeStruct((S,), jnp.float32),
        ],
        scratch_types=[
            pltpu.VMEM((_CHUNK,), jnp.int32),
            pltpu.VMEM((_CHUNK, D2), jnp.float32),
            pltpu.VMEM((_CHUNK,), jnp.float32),
            pltpu.VMEM((_CHUNK,), jnp.float32),
            pltpu.VMEM((_CHUNK,), jnp.float32),
            pltpu.SemaphoreType.DMA,
        ],
        compiler_params=pltpu.CompilerParams(use_tc_tiling_on_sc=False),
    )
    def gather_kernel(w2_hbm, bias_hbm, nlp_hbm, target_hbm, samples_hbm,
                      trows_hbm, srows_hbm, toff_hbm, soff_hbm,
                      idx_v, rows_v, b_v, n_v, o_v, sem):
        wid = lax.axis_index("s") * _NC + lax.axis_index("c")

        def do_chunk(idx_hbm, rows_out, off_out, base):
            pltpu.sync_copy(idx_hbm.at[pl.ds(base, _CHUNK)], idx_v)
            pltpu.async_copy(w2_hbm.at[idx_v], rows_v, sem).wait()
            pltpu.async_copy(bias_hbm.at[idx_v], b_v, sem).wait()
            pltpu.async_copy(nlp_hbm.at[idx_v], n_v, sem).wait()
            for k in range(_CHUNK // 16):
                sl = pl.ds(k * 16, 16)
                o_v[sl] = b_v[sl] - (n_v[sl] + log_ns)
            pltpu.sync_copy(rows_v, rows_out.at[pl.ds(base, _CHUNK)])
            pltpu.sync_copy(o_v, off_out.at[pl.ds(base, _CHUNK)])

        for c in range(B // (_NW * _CHUNK)):
            do_chunk(target_hbm, trows_hbm, toff_hbm,
                     wid * (B // _NW) + c * _CHUNK)
        for c in range(S // (_NW * _CHUNK)):
            do_chunk(samples_hbm, srows_hbm, soff_hbm,
                     wid * (S // _NW) + c * _CHUNK)

    return gather_kernel(w2, bias, nlp, target, samples)


_LOG2E = 1.4426950408889634
_LN2 = 0.6931471805599453


def _softplus_neg_abs(x):
    p = jnp.exp2(jnp.abs(x) * (-_LOG2E))
    return _LN2 * jnp.log2(1.0 + p)


def _tc_loss_T_body(se_ref, dt_ref, tt_ref, to_ref, so_ref, out_ref):
    j = pl.program_id(0)
    D = dt_ref.shape[0]
    mm = lax.dot_general(se_ref[...][:, :D], dt_ref[...],
                         dimension_numbers=(((1,), (0,)), ((), ())),
                         preferred_element_type=jnp.float32)
    sl = mm + so_ref[...]
    out_ref[...] = jnp.maximum(sl, 0.0) + _softplus_neg_abs(sl)

    @pl.when(j == 0)
    def _():
        tl = jnp.sum(dt_ref[...] * tt_ref[0:D, :], axis=0, keepdims=True)
        tl = tl + to_ref[...]
        out_ref[0:1, :] = jnp.maximum(tl, 0.0) - tl + _softplus_neg_abs(tl)


def _tc_loss_T(dataT, srows_p, soff_p, trowsT, toff_row, bt=512):
    D, B = dataT.shape
    D2 = srows_p.shape[1]
    S1 = srows_p.shape[0]
    return pl.pallas_call(
        _tc_loss_T_body,
        grid=(pl.cdiv(S1, bt),),
        in_specs=[
            pl.BlockSpec((bt, D2), lambda j: (j, 0)),
            pl.BlockSpec((D, B), lambda j: (0, 0)),
            pl.BlockSpec((D2, B), lambda j: (0, 0)),
            pl.BlockSpec((1, B), lambda j: (0, 0)),
            pl.BlockSpec((bt, 1), lambda j: (j, 0)),
        ],
        out_specs=pl.BlockSpec((bt, B), lambda j: (j, 0)),
        out_shape=jax.ShapeDtypeStruct((S1, B), jnp.float32),
        compiler_params=pltpu.CompilerParams(
            dimension_semantics=("arbitrary",),
        ),
    )(srows_p, dataT, trowsT, toff_row, soff_p)


def kernel(data, target, samples, weight, bias, noise_log_probs):
    B = data.shape[0]
    S = samples.shape[0]
    log_ns = math.log(S)
    w2 = _tc_expand(weight.T)
    trows, srows, toff, soff = _sc_gather(
        w2, bias, noise_log_probs,
        target.astype(jnp.int32), samples.astype(jnp.int32), log_ns)
    srows_p = jnp.pad(srows, ((1, 0), (0, 0)))
    soff_p = jnp.pad(soff, (1, 0)).reshape(S + 1, 1)
    out_T = _tc_loss_T(data.T, srows_p, soff_p, trows.T,
                       toff.reshape(1, B))
    return out_T.T


# speedup vs baseline: 2.4847x; 1.0489x over previous
import functools
import math

import jax
import jax.numpy as jnp
from jax import lax
from jax.experimental import pallas as pl
from jax.experimental.pallas import tpu as pltpu
from jax.experimental.pallas import tpu_sc as plsc

_NC = 2
_NS = 16
_NW = _NC * _NS
_CHUNK = 128

_PACK_NC = 16384
_PACK_H = _PACK_NC // 2


def _pack_rows(V):
    tail = V - (V // _PACK_NC) * _PACK_NC
    return (V // _PACK_NC) * _PACK_H + tail


def _pack_body(wt_ref, out_ref):
    D = wt_ref.shape[0]
    xt = wt_ref[...].T
    out_ref[:, :D] = xt[:_PACK_H]
    out_ref[:, D:] = xt[_PACK_H:]


def _tc_pack(weightT):
    D, V = weightT.shape
    return pl.pallas_call(
        _pack_body,
        grid=(pl.cdiv(V, _PACK_NC),),
        in_specs=[pl.BlockSpec((D, _PACK_NC), lambda j: (0, j))],
        out_specs=pl.BlockSpec((_PACK_H, 2 * D), lambda j: (j, 0)),
        out_shape=jax.ShapeDtypeStruct((_pack_rows(V), 2 * D), jnp.float32),
        compiler_params=pltpu.CompilerParams(
            dimension_semantics=("arbitrary",),
        ),
    )(weightT)


def _sc_gather(w2, bias, nlp, target, samples, log_ns):
    D2 = w2.shape[1]
    B = target.shape[0]
    S = samples.shape[0]
    sh_blk = _PACK_NC.bit_length() - 1
    sh_half = _PACK_H.bit_length() - 1
    mesh = plsc.VectorSubcoreMesh(core_axis_name="c", subcore_axis_name="s")

    @functools.partial(
        pl.kernel,
        mesh=mesh,
        out_type=[
            jax.ShapeDtypeStruct((B, D2), jnp.float32),
            jax.ShapeDtypeStruct((S, D2), jnp.float32),
            jax.ShapeDtypeStruct((B,), jnp.float32),
            jax.ShapeDtypeStruct((S,), jnp.float32),
            jax.ShapeDtypeStruct((B,), jnp.float32),
            jax.ShapeDtypeStruct((S,), jnp.float32),
        ],
        scratch_types=[
            pltpu.VMEM((_CHUNK,), jnp.int32),
            pltpu.VMEM((_CHUNK,), jnp.int32),
            pltpu.VMEM((_CHUNK, D2), jnp.float32),
            pltpu.VMEM((_CHUNK,), jnp.float32),
            pltpu.VMEM((_CHUNK,), jnp.float32),
            pltpu.VMEM((_CHUNK,), jnp.float32),
            pltpu.VMEM((_CHUNK,), jnp.float32),
            pltpu.SemaphoreType.DMA,
        ],
        compiler_params=pltpu.CompilerParams(use_tc_tiling_on_sc=False),
    )
    def gather_kernel(w2_hbm, bias_hbm, nlp_hbm, target_hbm, samples_hbm,
                      trows_hbm, srows_hbm, toff_hbm, soff_hbm,
                      tpar_hbm, spar_hbm,
                      idx_v, slot_v, rows_v, b_v, n_v, o_v, p_v, sem):
        wid = lax.axis_index("s") * _NC + lax.axis_index("c")

        def do_chunk(idx_hbm, rows_out, off_out, par_out, base):
            pltpu.sync_copy(idx_hbm.at[pl.ds(base, _CHUNK)], idx_v)
            for k in range(_CHUNK // 16):
                sl = pl.ds(k * 16, 16)
                c = idx_v[sl]
                slot_v[sl] = lax.bitwise_or(
                    lax.shift_left(lax.shift_right_logical(c, sh_blk),
                                   sh_half),
                    lax.bitwise_and(c, _PACK_H - 1))
                p_v[sl] = lax.convert_element_type(
                    lax.bitwise_and(lax.shift_right_logical(c, sh_half), 1),
                    jnp.float32)
            pltpu.async_copy(w2_hbm.at[slot_v], rows_v, sem).wait()
            pltpu.async_copy(bias_hbm.at[idx_v], b_v, sem).wait()
            pltpu.async_copy(nlp_hbm.at[idx_v], n_v, sem).wait()
            for k in range(_CHUNK // 16):
                sl = pl.ds(k * 16, 16)
                o_v[sl] = b_v[sl] - (n_v[sl] + log_ns)
            pltpu.sync_copy(rows_v, rows_out.at[pl.ds(base, _CHUNK)])
            pltpu.sync_copy(o_v, off_out.at[pl.ds(base, _CHUNK)])
            pltpu.sync_copy(p_v, par_out.at[pl.ds(base, _CHUNK)])

        for c in range(B // (_NW * _CHUNK)):
            do_chunk(target_hbm, trows_hbm, toff_hbm, tpar_hbm,
                     wid * (B // _NW) + c * _CHUNK)
        for c in range(S // (_NW * _CHUNK)):
            do_chunk(samples_hbm, srows_hbm, soff_hbm, spar_hbm,
                     wid * (S // _NW) + c * _CHUNK)

    return gather_kernel(w2, bias, nlp, target, samples)


_LOG2E = 1.4426950408889634
_LN2 = 0.6931471805599453


def _softplus_neg_abs(x):
    p = jnp.exp2(jnp.abs(x) * (-_LOG2E))
    return _LN2 * jnp.log2(1.0 + p)


def _tc_loss_T_body(se_ref, dt_ref, tt_ref, to_ref, tp_ref, so_ref, sp_ref,
                    out_ref):
    j = pl.program_id(0)
    D = dt_ref.shape[0]
    se = se_ref[...]
    emb = jnp.where(sp_ref[...] < 0.5, se[:, :D], se[:, D:])
    mm = lax.dot_general(emb, dt_ref[...],
                         dimension_numbers=(((1,), (0,)), ((), ())),
                         preferred_element_type=jnp.float32)
    sl = mm + so_ref[...]
    out_ref[...] = jnp.maximum(sl, 0.0) + _softplus_neg_abs(sl)

    @pl.when(j == 0)
    def _():
        tt = tt_ref[...]
        temb = jnp.where(tp_ref[...] < 0.5, tt[:D, :], tt[D:, :])
        tl = jnp.sum(dt_ref[...] * temb, axis=0, keepdims=True)
        tl = tl + to_ref[...]
        out_ref[0:1, :] = jnp.maximum(tl, 0.0) - tl + _softplus_neg_abs(tl)


def _tc_loss_T(dataT, srows_p, soff_p, spar_p, trowsT, toff_row, tpar_row,
               bt=512):
    D, B = dataT.shape
    D2 = srows_p.shape[1]
    S1 = srows_p.shape[0]
    return pl.pallas_call(
        _tc_loss_T_body,
        grid=(pl.cdiv(S1, bt),),
        in_specs=[
            pl.BlockSpec((bt, D2), lambda j: (j, 0)),
            pl.BlockSpec((D, B), lambda j: (0, 0)),
            pl.BlockSpec((D2, B), lambda j: (0, 0)),
            pl.BlockSpec((1, B), lambda j: (0, 0)),
            pl.BlockSpec((1, B), lambda j: (0, 0)),
            pl.BlockSpec((bt, 1), lambda j: (j, 0)),
            pl.BlockSpec((bt, 1), lambda j: (j, 0)),
        ],
        out_specs=pl.BlockSpec((bt, B), lambda j: (j, 0)),
        out_shape=jax.ShapeDtypeStruct((S1, B), jnp.float32),
        compiler_params=pltpu.CompilerParams(
            dimension_semantics=("arbitrary",),
        ),
    )(srows_p, dataT, trowsT, toff_row, tpar_row, soff_p, spar_p)


def kernel(data, target, samples, weight, bias, noise_log_probs):
    B = data.shape[0]
    S = samples.shape[0]
    log_ns = math.log(S)
    w2 = _tc_pack(weight.T)
    trows, srows, toff, soff, tpar, spar = _sc_gather(
        w2, bias, noise_log_probs,
        target.astype(jnp.int32), samples.astype(jnp.int32), log_ns)
    srows_p = jnp.pad(srows, ((1, 0), (0, 0)))
    soff_p = jnp.pad(soff, (1, 0)).reshape(S + 1, 1)
    spar_p = jnp.pad(spar, (1, 0)).reshape(S + 1, 1)
    out_T = _tc_loss_T(data.T, srows_p, soff_p, spar_p, trows.T,
                       toff.reshape(1, B), tpar.reshape(1, B))
    return out_T.T
